# baseline (speedup 1.0000x reference)
import jax
import jax.numpy as jnp
from jax.experimental import pallas as pl
from jax.experimental.pallas import tpu as pltpu

B = 1024
D = 1024
H = 16
S = 256
DH = D // H
EPS = 1e-8

BB = 256


def _mmt(a, b):
    return jax.lax.dot_general(a, b, (((1,), (1,)), ((), ())),
                               preferred_element_type=jnp.float32)


def _mm(a, b):
    return jax.lax.dot_general(a, b, (((1,), (0,)), ((), ())),
                               preferred_element_type=jnp.float32)


def _fused(q_ref, c_ref, fk_ref, fv_ref, dk_ref, dv_ref,
           wq_ref, bq_ref, wg_ref, bg_ref, mix_ref,
           wc1_ref, bc1_ref, wc2_ref, bc2_ref,
           wo_ref, bo_ref, out_ref,
           wq_st, wg_st, wc1_st, wo_st,
           wqb, wgb, wc1b, wob, fkn_s, dkn_s, fva, dva, m_s,
           sem_q, sem_g, sem_c1, sem_o):
    bf = jnp.bfloat16
    first = pl.program_id(0) == 0

    @pl.when(first)
    def _start_and_prep():
        pltpu.make_async_copy(wg_ref, wg_st, sem_g).start()
        pltpu.make_async_copy(wc1_ref, wc1_st, sem_c1).start()
        pltpu.make_async_copy(wq_ref, wq_st, sem_q).start()
        pltpu.make_async_copy(wo_ref, wo_st, sem_o).start()
        fk = fk_ref[...]
        dk = dk_ref[...]
        fkn_s[...] = (fk * (1.0 / (jnp.sqrt(
            jnp.sum(fk * fk, axis=2, keepdims=True)) + EPS))).astype(bf)
        dkn_s[...] = (dk * (1.0 / (jnp.sqrt(
            jnp.sum(dk * dk, axis=2, keepdims=True)) + EPS))).astype(bf)
        ones_col = jnp.ones((H, S, 1), dtype=bf)
        fva[...] = jnp.concatenate([fv_ref[...].astype(bf), ones_col], axis=2)
        dva[...] = jnp.concatenate([dv_ref[...].astype(bf), ones_col], axis=2)
        di = jax.lax.broadcasted_iota(jnp.int32, (D, H), 0) // DH
        hi = jax.lax.broadcasted_iota(jnp.int32, (D, H), 1)
        m_s[...] = (di == hi).astype(jnp.float32)

    x = q_ref[...].astype(bf)
    c = c_ref[...].astype(bf)

    @pl.when(first)
    def _land_wg():
        pltpu.make_async_copy(wg_ref, wg_st, sem_g).wait()
        wgb[...] = wg_st[...].astype(bf)

    g = jnp.tanh(_mmt(c, wgb[...]) + bg_ref[...])
    gate = jnp.mean(g, axis=1, keepdims=True)
    mix = jax.nn.sigmoid(mix_ref[0, 0] + gate)

    @pl.when(first)
    def _land_wc1():
        pltpu.make_async_copy(wc1_ref, wc1_st, sem_c1).wait()
        wc1b[...] = wc1_st[...].astype(bf)

    c1 = jnp.tanh(_mmt(c, wc1b[...]) + bc1_ref[...])
    conf = jax.nn.sigmoid(
        jnp.sum(c1 * wc2_ref[...], axis=1, keepdims=True) + bc2_ref[0, 0])

    sf = mix * conf
    sd = (1.0 - mix) * conf

    @pl.when(first)
    def _land_wq():
        pltpu.make_async_copy(wq_ref, wq_st, sem_q).wait()
        wqb[...] = wq_st[...].astype(bf)

    q = _mmt(x, wqb[...]) + bq_ref[...]
    n2 = _mm(q * q, m_s[...])
    inv = 1.0 / (jnp.sqrt(n2) + EPS)
    inv_exp = _mmt(inv, m_s[...])
    qs = (q * inv_exp).astype(bf)

    parts = []
    for h in range(H):
        qh = qs[:, h * DH:(h + 1) * DH]
        ef = jnp.exp(_mmt(qh, fkn_s[h])).astype(bf)
        ed = jnp.exp(_mmt(qh, dkn_s[h])).astype(bf)
        vf = _mm(ef, fva[h])
        vd = _mm(ed, dva[h])
        scf = sf / vf[:, DH:DH + 1]
        scd = sd / vd[:, DH:DH + 1]
        parts.append((vf[:, :DH] * scf + vd[:, :DH] * scd).astype(bf))
    pre = jnp.concatenate(parts, axis=1)

    @pl.when(first)
    def _land_wo():
        pltpu.make_async_copy(wo_ref, wo_st, sem_o).wait()
        wob[...] = wo_st[...].astype(bf)

    out_ref[...] = _mmt(pre, wob[...]) + bo_ref[...]


def kernel(query, context, fast_keys, fast_vals, deep_keys, deep_vals,
           W_q, b_q, W_gate, b_gate, mix_logit, W_c1, b_c1, W_c2, b_c2,
           W_out, b_out):
    grid = (B // BB,)
    row_spec = pl.BlockSpec((BB, D), lambda i: (i, 0))
    hbm_spec = pl.BlockSpec(memory_space=pltpu.MemorySpace.HBM)

    def full(shape):
        return pl.BlockSpec(shape, lambda i: (0,) * len(shape))

    bf = jnp.bfloat16
    out = pl.pallas_call(
        _fused,
        grid=grid,
        in_specs=[
            row_spec,
            row_spec,
            full((H, S, DH)),
            full((H, S, DH)),
            full((H, S, DH)),
            full((H, S, DH)),
            hbm_spec,
            full((1, D)),
            hbm_spec,
            full((1, D)),
            full((1, 1)),
            hbm_spec,
            full((1, D)),
            full((1, D)),
            full((1, 1)),
            hbm_spec,
            full((1, D)),
        ],
        out_specs=row_spec,
        out_shape=jax.ShapeDtypeStruct((B, D), jnp.float32),
        scratch_shapes=[
            pltpu.VMEM((D, D), jnp.float32),
            pltpu.VMEM((D, D), jnp.float32),
            pltpu.VMEM((D, D), jnp.float32),
            pltpu.VMEM((D, D), jnp.float32),
            pltpu.VMEM((D, D), bf),
            pltpu.VMEM((D, D), bf),
            pltpu.VMEM((D, D), bf),
            pltpu.VMEM((D, D), bf),
            pltpu.VMEM((H, S, DH), bf),
            pltpu.VMEM((H, S, DH), bf),
            pltpu.VMEM((H, S, DH + 1), bf),
            pltpu.VMEM((H, S, DH + 1), bf),
            pltpu.VMEM((D, H), jnp.float32),
            pltpu.SemaphoreType.DMA,
            pltpu.SemaphoreType.DMA,
            pltpu.SemaphoreType.DMA,
            pltpu.SemaphoreType.DMA,
        ],
    )(
        query, context,
        fast_keys.reshape(H, S, DH), fast_vals.reshape(H, S, DH),
        deep_keys.reshape(H, S, DH), deep_vals.reshape(H, S, DH),
        W_q, b_q.reshape(1, D),
        W_gate, b_gate.reshape(1, D),
        mix_logit.reshape(1, 1),
        W_c1, b_c1.reshape(1, D),
        W_c2, b_c2.reshape(1, 1),
        W_out, b_out.reshape(1, D),
    )
    return out


# speedup vs baseline: 1.1483x; 1.1483x over previous
import jax
import jax.numpy as jnp
from jax.experimental import pallas as pl
from jax.experimental.pallas import tpu as pltpu

B = 1024
D = 1024
H = 16
S = 256
DH = D // H
EPS = 1e-8

BB = 512


def _mmt(a, b):
    return jax.lax.dot_general(a, b, (((1,), (1,)), ((), ())),
                               preferred_element_type=jnp.float32)


def _mm(a, b):
    return jax.lax.dot_general(a, b, (((1,), (0,)), ((), ())),
                               preferred_element_type=jnp.float32)


def _fused(q_ref, c_ref, fk_ref, fv_ref, dk_ref, dv_ref,
           wq_ref, bq_ref, wg_ref, bg_ref, mix_ref,
           wc1_ref, bc1_ref, wc2_ref, bc2_ref,
           wo_ref, bo_ref, out_ref,
           wq_st, wg_st, wc1_st, wo_st,
           wqb, wgb, wc1b, wob, fkn_s, dkn_s, fva, dva, m_s,
           sem_q, sem_g, sem_c1, sem_o):
    bf = jnp.bfloat16
    first = pl.program_id(0) == 0

    @pl.when(first)
    def _start_and_prep():
        pltpu.make_async_copy(wg_ref, wg_st, sem_g).start()
        pltpu.make_async_copy(wc1_ref, wc1_st, sem_c1).start()
        pltpu.make_async_copy(wq_ref, wq_st, sem_q).start()
        pltpu.make_async_copy(wo_ref, wo_st, sem_o).start()
        fk = fk_ref[...]
        dk = dk_ref[...]
        fkn_s[...] = (fk * (1.0 / (jnp.sqrt(
            jnp.sum(fk * fk, axis=2, keepdims=True)) + EPS))).astype(bf)
        dkn_s[...] = (dk * (1.0 / (jnp.sqrt(
            jnp.sum(dk * dk, axis=2, keepdims=True)) + EPS))).astype(bf)
        ones_col = jnp.ones((H, S, 1), dtype=bf)
        fva[...] = jnp.concatenate([fv_ref[...].astype(bf), ones_col], axis=2)
        dva[...] = jnp.concatenate([dv_ref[...].astype(bf), ones_col], axis=2)
        di = jax.lax.broadcasted_iota(jnp.int32, (D, H), 0) // DH
        hi = jax.lax.broadcasted_iota(jnp.int32, (D, H), 1)
        m_s[...] = (di == hi).astype(jnp.float32)

    x = q_ref[...].astype(bf)
    c = c_ref[...].astype(bf)

    @pl.when(first)
    def _land_wg():
        pltpu.make_async_copy(wg_ref, wg_st, sem_g).wait()
        wgb[...] = wg_st[...].astype(bf)

    g = jnp.tanh(_mmt(c, wgb[...]) + bg_ref[...])
    gate = jnp.mean(g, axis=1, keepdims=True)
    mix = jax.nn.sigmoid(mix_ref[0, 0] + gate)

    @pl.when(first)
    def _land_wc1():
        pltpu.make_async_copy(wc1_ref, wc1_st, sem_c1).wait()
        wc1b[...] = wc1_st[...].astype(bf)

    c1 = jnp.tanh(_mmt(c, wc1b[...]) + bc1_ref[...])
    conf = jax.nn.sigmoid(
        jnp.sum(c1 * wc2_ref[...], axis=1, keepdims=True) + bc2_ref[0, 0])

    sf = mix * conf
    sd = (1.0 - mix) * conf

    @pl.when(first)
    def _land_wq():
        pltpu.make_async_copy(wq_ref, wq_st, sem_q).wait()
        wqb[...] = wq_st[...].astype(bf)

    q = _mmt(x, wqb[...]) + bq_ref[...]
    n2 = _mm(q * q, m_s[...])
    inv = 1.0 / (jnp.sqrt(n2) + EPS)
    inv_exp = _mmt(inv, m_s[...])
    qs = (q * inv_exp).astype(bf)

    parts = []
    for h in range(H):
        qh = qs[:, h * DH:(h + 1) * DH]
        ef = jnp.exp(_mmt(qh, fkn_s[h])).astype(bf)
        ed = jnp.exp(_mmt(qh, dkn_s[h])).astype(bf)
        vf = _mm(ef, fva[h])
        vd = _mm(ed, dva[h])
        scf = sf / vf[:, DH:DH + 1]
        scd = sd / vd[:, DH:DH + 1]
        parts.append((vf[:, :DH] * scf + vd[:, :DH] * scd).astype(bf))
    pre = jnp.concatenate(parts, axis=1)

    @pl.when(first)
    def _land_wo():
        pltpu.make_async_copy(wo_ref, wo_st, sem_o).wait()
        wob[...] = wo_st[...].astype(bf)

    out_ref[...] = _mmt(pre, wob[...]) + bo_ref[...]


def kernel(query, context, fast_keys, fast_vals, deep_keys, deep_vals,
           W_q, b_q, W_gate, b_gate, mix_logit, W_c1, b_c1, W_c2, b_c2,
           W_out, b_out):
    grid = (B // BB,)
    row_spec = pl.BlockSpec((BB, D), lambda i: (i, 0))
    hbm_spec = pl.BlockSpec(memory_space=pltpu.MemorySpace.HBM)

    def full(shape):
        return pl.BlockSpec(shape, lambda i: (0,) * len(shape))

    bf = jnp.bfloat16
    out = pl.pallas_call(
        _fused,
        grid=grid,
        in_specs=[
            row_spec,
            row_spec,
            full((H, S, DH)),
            full((H, S, DH)),
            full((H, S, DH)),
            full((H, S, DH)),
            hbm_spec,
            full((1, D)),
            hbm_spec,
            full((1, D)),
            full((1, 1)),
            hbm_spec,
            full((1, D)),
            full((1, D)),
            full((1, 1)),
            hbm_spec,
            full((1, D)),
        ],
        out_specs=row_spec,
        out_shape=jax.ShapeDtypeStruct((B, D), jnp.float32),
        scratch_shapes=[
            pltpu.VMEM((D, D), jnp.float32),
            pltpu.VMEM((D, D), jnp.float32),
            pltpu.VMEM((D, D), jnp.float32),
            pltpu.VMEM((D, D), jnp.float32),
            pltpu.VMEM((D, D), bf),
            pltpu.VMEM((D, D), bf),
            pltpu.VMEM((D, D), bf),
            pltpu.VMEM((D, D), bf),
            pltpu.VMEM((H, S, DH), bf),
            pltpu.VMEM((H, S, DH), bf),
            pltpu.VMEM((H, S, DH + 1), bf),
            pltpu.VMEM((H, S, DH + 1), bf),
            pltpu.VMEM((D, H), jnp.float32),
            pltpu.SemaphoreType.DMA,
            pltpu.SemaphoreType.DMA,
            pltpu.SemaphoreType.DMA,
            pltpu.SemaphoreType.DMA,
        ],
    )(
        query, context,
        fast_keys.reshape(H, S, DH), fast_vals.reshape(H, S, DH),
        deep_keys.reshape(H, S, DH), deep_vals.reshape(H, S, DH),
        W_q, b_q.reshape(1, D),
        W_gate, b_gate.reshape(1, D),
        mix_logit.reshape(1, 1),
        W_c1, b_c1.reshape(1, D),
        W_c2, b_c2.reshape(1, 1),
        W_out, b_out.reshape(1, D),
    )
    return out


# speedup vs baseline: 1.1547x; 1.0056x over previous
import jax
import jax.numpy as jnp
from jax.experimental import pallas as pl
from jax.experimental.pallas import tpu as pltpu

B = 1024
D = 1024
H = 16
S = 256
DH = D // H
EPS = 1e-8

BB = 1024


def _mmt(a, b):
    return jax.lax.dot_general(a, b, (((1,), (1,)), ((), ())),
                               preferred_element_type=jnp.float32)


def _mm(a, b):
    return jax.lax.dot_general(a, b, (((1,), (0,)), ((), ())),
                               preferred_element_type=jnp.float32)


def _fused(q_ref, c_ref, fk_ref, fv_ref, dk_ref, dv_ref,
           wq_ref, bq_ref, wg_ref, bg_ref, mix_ref,
           wc1_ref, bc1_ref, wc2_ref, bc2_ref,
           wo_ref, bo_ref, out_ref,
           wq_st, wg_st, wc1_st, wo_st,
           wqb, wgb, wc1b, wob, fkn_s, dkn_s, fva, dva, m_s,
           sem_q, sem_g, sem_c1, sem_o):
    bf = jnp.bfloat16
    first = pl.program_id(0) == 0

    @pl.when(first)
    def _start_and_prep():
        pltpu.make_async_copy(wg_ref, wg_st, sem_g).start()
        pltpu.make_async_copy(wc1_ref, wc1_st, sem_c1).start()
        pltpu.make_async_copy(wq_ref, wq_st, sem_q).start()
        pltpu.make_async_copy(wo_ref, wo_st, sem_o).start()
        fk = fk_ref[...]
        dk = dk_ref[...]
        fkn_s[...] = (fk * (1.0 / (jnp.sqrt(
            jnp.sum(fk * fk, axis=2, keepdims=True)) + EPS))).astype(bf)
        dkn_s[...] = (dk * (1.0 / (jnp.sqrt(
            jnp.sum(dk * dk, axis=2, keepdims=True)) + EPS))).astype(bf)
        ones_col = jnp.ones((H, S, 1), dtype=bf)
        fva[...] = jnp.concatenate([fv_ref[...].astype(bf), ones_col], axis=2)
        dva[...] = jnp.concatenate([dv_ref[...].astype(bf), ones_col], axis=2)
        di = jax.lax.broadcasted_iota(jnp.int32, (D, H), 0) // DH
        hi = jax.lax.broadcasted_iota(jnp.int32, (D, H), 1)
        m_s[...] = (di == hi).astype(jnp.float32)

    x = q_ref[...].astype(bf)
    c = c_ref[...].astype(bf)

    @pl.when(first)
    def _land_wg():
        pltpu.make_async_copy(wg_ref, wg_st, sem_g).wait()
        wgb[...] = wg_st[...].astype(bf)

    g = jnp.tanh(_mmt(c, wgb[...]) + bg_ref[...])
    gate = jnp.mean(g, axis=1, keepdims=True)
    mix = jax.nn.sigmoid(mix_ref[0, 0] + gate)

    @pl.when(first)
    def _land_wc1():
        pltpu.make_async_copy(wc1_ref, wc1_st, sem_c1).wait()
        wc1b[...] = wc1_st[...].astype(bf)

    c1 = jnp.tanh(_mmt(c, wc1b[...]) + bc1_ref[...])
    conf = jax.nn.sigmoid(
        jnp.sum(c1 * wc2_ref[...], axis=1, keepdims=True) + bc2_ref[0, 0])

    sf = mix * conf
    sd = (1.0 - mix) * conf

    @pl.when(first)
    def _land_wq():
        pltpu.make_async_copy(wq_ref, wq_st, sem_q).wait()
        wqb[...] = wq_st[...].astype(bf)

    q = _mmt(x, wqb[...]) + bq_ref[...]
    n2 = _mm(q * q, m_s[...])
    inv = 1.0 / (jnp.sqrt(n2) + EPS)
    inv_exp = _mmt(inv, m_s[...])
    qs = (q * inv_exp).astype(bf)

    parts = []
    for h in range(H):
        qh = qs[:, h * DH:(h + 1) * DH]
        ef = jnp.exp(_mmt(qh, fkn_s[h])).astype(bf)
        ed = jnp.exp(_mmt(qh, dkn_s[h])).astype(bf)
        vf = _mm(ef, fva[h])
        vd = _mm(ed, dva[h])
        scf = sf / vf[:, DH:DH + 1]
        scd = sd / vd[:, DH:DH + 1]
        parts.append((vf[:, :DH] * scf + vd[:, :DH] * scd).astype(bf))
    pre = jnp.concatenate(parts, axis=1)

    @pl.when(first)
    def _land_wo():
        pltpu.make_async_copy(wo_ref, wo_st, sem_o).wait()
        wob[...] = wo_st[...].astype(bf)

    out_ref[...] = _mmt(pre, wob[...]) + bo_ref[...]


def kernel(query, context, fast_keys, fast_vals, deep_keys, deep_vals,
           W_q, b_q, W_gate, b_gate, mix_logit, W_c1, b_c1, W_c2, b_c2,
           W_out, b_out):
    grid = (B // BB,)
    row_spec = pl.BlockSpec((BB, D), lambda i: (i, 0))
    hbm_spec = pl.BlockSpec(memory_space=pltpu.MemorySpace.HBM)

    def full(shape):
        return pl.BlockSpec(shape, lambda i: (0,) * len(shape))

    bf = jnp.bfloat16
    out = pl.pallas_call(
        _fused,
        grid=grid,
        in_specs=[
            row_spec,
            row_spec,
            full((H, S, DH)),
            full((H, S, DH)),
            full((H, S, DH)),
            full((H, S, DH)),
            hbm_spec,
            full((1, D)),
            hbm_spec,
            full((1, D)),
            full((1, 1)),
            hbm_spec,
            full((1, D)),
            full((1, D)),
            full((1, 1)),
            hbm_spec,
            full((1, D)),
        ],
        out_specs=row_spec,
        out_shape=jax.ShapeDtypeStruct((B, D), jnp.float32),
        compiler_params=pltpu.CompilerParams(
            vmem_limit_bytes=128 * 1024 * 1024),
        scratch_shapes=[
            pltpu.VMEM((D, D), jnp.float32),
            pltpu.VMEM((D, D), jnp.float32),
            pltpu.VMEM((D, D), jnp.float32),
            pltpu.VMEM((D, D), jnp.float32),
            pltpu.VMEM((D, D), bf),
            pltpu.VMEM((D, D), bf),
            pltpu.VMEM((D, D), bf),
            pltpu.VMEM((D, D), bf),
            pltpu.VMEM((H, S, DH), bf),
            pltpu.VMEM((H, S, DH), bf),
            pltpu.VMEM((H, S, DH + 1), bf),
            pltpu.VMEM((H, S, DH + 1), bf),
            pltpu.VMEM((D, H), jnp.float32),
            pltpu.SemaphoreType.DMA,
            pltpu.SemaphoreType.DMA,
            pltpu.SemaphoreType.DMA,
            pltpu.SemaphoreType.DMA,
        ],
    )(
        query, context,
        fast_keys.reshape(H, S, DH), fast_vals.reshape(H, S, DH),
        deep_keys.reshape(H, S, DH), deep_vals.reshape(H, S, DH),
        W_q, b_q.reshape(1, D),
        W_gate, b_gate.reshape(1, D),
        mix_logit.reshape(1, 1),
        W_c1, b_c1.reshape(1, D),
        W_c2, b_c2.reshape(1, 1),
        W_out, b_out.reshape(1, D),
    )
    return out
